# baseline (speedup 1.0000x reference)
import jax
import jax.numpy as jnp
from jax.experimental import pallas as pl
from jax.experimental.pallas import tpu as pltpu

D = 128
N = 32
BATCH = 4
S = 256
E = 256
NCLS = 10
CIN = N * D
F32 = jnp.float32


def _relu(x):
    return jnp.maximum(x, 0.0)


def _dot(a, b):
    return jnp.dot(a, b, preferred_element_type=F32)


def _dot_hi(a, b):
    return jnp.dot(a, b, preferred_element_type=F32,
                   precision=jax.lax.Precision.HIGHEST)


def _shift_down(p):
    return jnp.concatenate([jnp.zeros_like(p[:1]), p[:-1]], axis=0)


def _shift_up(p):
    return jnp.concatenate([p[1:], jnp.zeros_like(p[:1])], axis=0)


def _conv_s1(x, w0, w1, w2, dot=None):
    dot = dot or _dot
    return _shift_down(dot(x, w0)) + dot(x, w1) + _shift_up(dot(x, w2))


def _conv_s2(x, w0, w1, w2, dot=None):
    dot = dot or _dot
    L, C = x.shape
    v = x.reshape(L // 2, 2 * C)
    w01 = jnp.concatenate([w0, w1], axis=0)
    p01 = dot(v, w01)
    p2 = dot(v[:, :C], w2)
    return p01 + _shift_up(p2)


def _up2(x):
    L, C = x.shape
    return jnp.broadcast_to(x[:, None, :], (L, 2, C)).reshape(2 * L, C)


def _body(xt_ref, edg_ref, w0_ref, b0_ref, we_ref, be_ref,
          k1_ref, k2_ref, kd1_ref, kd2_ref, ko_ref, out_ref):
    src = edg_ref[0:1, :]
    dst = edg_ref[1:2, :]
    ni = jax.lax.broadcasted_iota(jnp.int32, (N, E), 0)
    dst_oh = (ni == dst).astype(F32)
    src_oh = (ni == src).astype(F32)
    A = jax.lax.dot_general(dst_oh, src_oh, (((1,), (1,)), ((), ())),
                            preferred_element_type=F32)
    r = jax.lax.broadcasted_iota(jnp.int32, (N, N), 0)
    c = jax.lax.broadcasted_iota(jnp.int32, (N, N), 1)
    M = A + (r == c).astype(F32)

    h = xt_ref[0].reshape(N * S, 2)
    z = _dot_hi(h, w0_ref[...])
    t = _dot_hi(M, z.reshape(N, S * D))
    h = _relu(t.reshape(N * S, D) + b0_ref[...])
    for i in range(4):
        z = _dot_hi(h, we_ref[i])
        t = _dot_hi(M, z.reshape(N, S * D))
        h = _relu(t.reshape(N * S, D) + be_ref[i:i + 1, :])

    hc = jnp.swapaxes(h.reshape(N, S, D), 0, 1).reshape(S, CIN)

    e1 = _relu(_conv_s2(hc, k1_ref[0], k1_ref[1], k1_ref[2], dot=_dot_hi))
    e2 = _relu(_conv_s2(e1, k2_ref[0], k2_ref[1], k2_ref[2], dot=_dot_hi))
    u1 = _up2(e2)
    d1 = _relu(_conv_s1(u1, kd1_ref[0, :256], kd1_ref[1, :256], kd1_ref[2, :256], dot=_dot_hi)
               + _conv_s1(e1, kd1_ref[0, 256:], kd1_ref[1, 256:], kd1_ref[2, 256:], dot=_dot_hi))
    u2 = _up2(d1)
    d2 = _relu(_conv_s1(u2, kd2_ref[0, :256], kd2_ref[1, :256], kd2_ref[2, :256])
               + _conv_s1(hc, kd2_ref[0, 256:], kd2_ref[1, 256:], kd2_ref[2, 256:]))
    out_ref[0] = _dot(d2, ko_ref[...])


def kernel(x_, edges, W0, b0, W_enc, b_enc, K1, K2, Kd1, Kd2, Kout):
    xt = jnp.transpose(x_, (0, 2, 1, 3))
    b0r = b0.reshape(1, D)
    k1t = jnp.transpose(K1, (2, 1, 0))
    k2t = jnp.transpose(K2, (2, 1, 0))
    kd1t = jnp.transpose(Kd1, (2, 1, 0))
    kd2t = jnp.transpose(Kd2, (2, 1, 0))
    kot = Kout[:, :, 0].T

    whole = lambda shape: pl.BlockSpec(shape, lambda b: (0,) * len(shape))
    out = pl.pallas_call(
        _body,
        grid=(BATCH,),
        in_specs=[
            pl.BlockSpec((1, N, S, 2), lambda b: (b, 0, 0, 0)),
            whole((2, E)),
            whole((2, D)),
            whole((1, D)),
            whole((4, D, D)),
            whole((4, D)),
            whole((3, CIN, 256)),
            whole((3, 256, 256)),
            whole((3, 512, 256)),
            whole((3, 256 + CIN, 256)),
            whole((256, NCLS)),
        ],
        out_specs=pl.BlockSpec((1, S, NCLS), lambda b: (b, 0, 0)),
        out_shape=jax.ShapeDtypeStruct((BATCH, S, NCLS), F32),
        compiler_params=pltpu.CompilerParams(
            vmem_limit_bytes=100 * 1024 * 1024,
        ),
    )(xt, edges, W0, b0r, W_enc, b_enc, k1t, k2t, kd1t, kd2t, kot)
    return jnp.transpose(out, (0, 2, 1))


# speedup vs baseline: 1.0735x; 1.0735x over previous
import jax
import jax.numpy as jnp
from jax.experimental import pallas as pl
from jax.experimental.pallas import tpu as pltpu

D = 128
N = 32
BATCH = 4
S = 256
E = 256
NCLS = 10
CIN = N * D
F32 = jnp.float32


def _relu(x):
    return jnp.maximum(x, 0.0)


def _dot(a, b):
    return jnp.dot(a, b, preferred_element_type=F32)


def _dot_hi(a, b):
    return jnp.dot(a, b, preferred_element_type=F32,
                   precision=jax.lax.Precision.HIGHEST)


def _shift_down(p):
    return jnp.concatenate([jnp.zeros_like(p[:1]), p[:-1]], axis=0)


def _shift_up(p):
    return jnp.concatenate([p[1:], jnp.zeros_like(p[:1])], axis=0)


def _conv_s1(x, w0, w1, w2, dot=None):
    dot = dot or _dot
    return _shift_down(dot(x, w0)) + dot(x, w1) + _shift_up(dot(x, w2))


def _conv_s2(x, w0, w1, w2, dot=None):
    dot = dot or _dot
    L, C = x.shape
    v = x.reshape(L // 2, 2 * C)
    w01 = jnp.concatenate([w0, w1], axis=0)
    p01 = dot(v, w01)
    p2 = dot(v[:, :C], w2)
    return p01 + _shift_up(p2)


def _up2(x):
    L, C = x.shape
    return jnp.broadcast_to(x[:, None, :], (L, 2, C)).reshape(2 * L, C)


def _body(xt_ref, edg_ref, w0_ref, b0_ref, we_ref, be_ref,
          k1_ref, k2_ref, kd1_ref, kd2_ref, ko_ref, out_ref):
    src = edg_ref[0:1, :]
    dst = edg_ref[1:2, :]
    ni = jax.lax.broadcasted_iota(jnp.int32, (N, E), 0)
    dst_oh = (ni == dst).astype(F32)
    src_oh = (ni == src).astype(F32)
    A = jax.lax.dot_general(dst_oh, src_oh, (((1,), (1,)), ((), ())),
                            preferred_element_type=F32)
    r = jax.lax.broadcasted_iota(jnp.int32, (N, N), 0)
    c = jax.lax.broadcasted_iota(jnp.int32, (N, N), 1)
    M = A + (r == c).astype(F32)

    h = xt_ref[0].reshape(N * S, 2)
    z = _dot_hi(h, w0_ref[...])
    t = _dot_hi(M, z.reshape(N, S * D))
    h = _relu(t.reshape(N * S, D) + b0_ref[...])
    for i in range(4):
        z = _dot_hi(h, we_ref[i])
        t = _dot_hi(M, z.reshape(N, S * D))
        h = _relu(t.reshape(N * S, D) + be_ref[i:i + 1, :])

    hc = jnp.swapaxes(h.reshape(N, S, D), 0, 1).reshape(S, CIN)

    e1 = _relu(_conv_s2(hc, k1_ref[0], k1_ref[1], k1_ref[2]))
    e2 = _relu(_conv_s2(e1, k2_ref[0], k2_ref[1], k2_ref[2]))
    u1 = _up2(e2)
    d1 = _relu(_conv_s1(u1, kd1_ref[0, :256], kd1_ref[1, :256], kd1_ref[2, :256])
               + _conv_s1(e1, kd1_ref[0, 256:], kd1_ref[1, 256:], kd1_ref[2, 256:]))
    u2 = _up2(d1)
    d2 = _relu(_conv_s1(u2, kd2_ref[0, :256], kd2_ref[1, :256], kd2_ref[2, :256])
               + _conv_s1(hc, kd2_ref[0, 256:], kd2_ref[1, 256:], kd2_ref[2, 256:]))
    out_ref[0] = _dot(d2, ko_ref[...])


def kernel(x_, edges, W0, b0, W_enc, b_enc, K1, K2, Kd1, Kd2, Kout):
    xt = jnp.transpose(x_, (0, 2, 1, 3))
    b0r = b0.reshape(1, D)
    k1t = jnp.transpose(K1, (2, 1, 0))
    k2t = jnp.transpose(K2, (2, 1, 0))
    kd1t = jnp.transpose(Kd1, (2, 1, 0))
    kd2t = jnp.transpose(Kd2, (2, 1, 0))
    kot = Kout[:, :, 0].T

    whole = lambda shape: pl.BlockSpec(shape, lambda b: (0,) * len(shape))
    out = pl.pallas_call(
        _body,
        grid=(BATCH,),
        in_specs=[
            pl.BlockSpec((1, N, S, 2), lambda b: (b, 0, 0, 0)),
            whole((2, E)),
            whole((2, D)),
            whole((1, D)),
            whole((4, D, D)),
            whole((4, D)),
            whole((3, CIN, 256)),
            whole((3, 256, 256)),
            whole((3, 512, 256)),
            whole((3, 256 + CIN, 256)),
            whole((256, NCLS)),
        ],
        out_specs=pl.BlockSpec((1, S, NCLS), lambda b: (b, 0, 0)),
        out_shape=jax.ShapeDtypeStruct((BATCH, S, NCLS), F32),
        compiler_params=pltpu.CompilerParams(
            vmem_limit_bytes=100 * 1024 * 1024,
        ),
    )(xt, edges, W0, b0r, W_enc, b_enc, k1t, k2t, kd1t, kd2t, kot)
    return jnp.transpose(out, (0, 2, 1))


# speedup vs baseline: 3.2021x; 2.9828x over previous
import jax
import jax.numpy as jnp
from jax.experimental import pallas as pl
from jax.experimental.pallas import tpu as pltpu

D = 128
N = 32
BATCH = 4
S = 256
E = 256
NCLS = 10
CIN = N * D
F32 = jnp.float32


def _relu(x):
    return jnp.maximum(x, 0.0)


def _dot(a, b):
    return jnp.dot(a, b, preferred_element_type=F32)


def _dot_hi(a, b):
    return jnp.dot(a, b, preferred_element_type=F32,
                   precision=jax.lax.Precision.HIGHEST)


def _shift_down(p):
    return jnp.concatenate([jnp.zeros_like(p[:1]), p[:-1]], axis=0)


def _shift_up(p):
    return jnp.concatenate([p[1:], jnp.zeros_like(p[:1])], axis=0)


def _conv_s1(x, w0, w1, w2, dot=None):
    dot = dot or _dot
    return _shift_down(dot(x, w0)) + dot(x, w1) + _shift_up(dot(x, w2))


def _conv_s2(x, w0, w1, w2, dot=None):
    dot = dot or _dot
    L, C = x.shape
    v = x.reshape(L // 2, 2 * C)
    w01 = jnp.concatenate([w0, w1], axis=0)
    p01 = dot(v, w01)
    p2 = dot(v[:, :C], w2)
    return p01 + _shift_up(p2)


def _up2(x):
    L, C = x.shape
    return jnp.broadcast_to(x[:, None, :], (L, 2, C)).reshape(2 * L, C)


def _body(xt_ref, edg_ref, w0_ref, b0_ref, we_ref, be_ref,
          k1_ref, k2_ref, kd1_ref, kd2_ref, ko_ref, out_ref):
    src = edg_ref[0:1, :]
    dst = edg_ref[1:2, :]
    ni = jax.lax.broadcasted_iota(jnp.int32, (N, E), 0)
    dst_oh = (ni == dst).astype(F32)
    src_oh = (ni == src).astype(F32)
    A = jax.lax.dot_general(dst_oh, src_oh, (((1,), (1,)), ((), ())),
                            preferred_element_type=F32)
    r = jax.lax.broadcasted_iota(jnp.int32, (N, N), 0)
    c = jax.lax.broadcasted_iota(jnp.int32, (N, N), 1)
    M = A + (r == c).astype(F32)

    def _wmul(h3, w):
        return jax.lax.dot_general(h3, w, (((2,), (0,)), ((), ())),
                                   preferred_element_type=F32)

    def _mmul(m, z3):
        return jax.lax.dot_general(m, z3, (((1,), (0,)), ((), ())),
                                   preferred_element_type=F32)

    h = xt_ref[0]
    h = _relu(_mmul(M, _wmul(h, w0_ref[...])) + b0_ref[...].reshape(1, 1, D))
    for i in range(4):
        h = _relu(_mmul(M, _wmul(h, we_ref[i])) + be_ref[i:i + 1, :].reshape(1, 1, D))
    h = h.reshape(N * S, D)

    hc = jnp.swapaxes(h.reshape(N, S, D), 0, 1).reshape(S, CIN)

    e1 = _relu(_conv_s2(hc, k1_ref[0], k1_ref[1], k1_ref[2]))
    e2 = _relu(_conv_s2(e1, k2_ref[0], k2_ref[1], k2_ref[2]))
    u1 = _up2(e2)
    d1 = _relu(_conv_s1(u1, kd1_ref[0, :256], kd1_ref[1, :256], kd1_ref[2, :256])
               + _conv_s1(e1, kd1_ref[0, 256:], kd1_ref[1, 256:], kd1_ref[2, 256:]))
    u2 = _up2(d1)
    d2 = _relu(_conv_s1(u2, kd2_ref[0, :256], kd2_ref[1, :256], kd2_ref[2, :256])
               + _conv_s1(hc, kd2_ref[0, 256:], kd2_ref[1, 256:], kd2_ref[2, 256:]))
    out_ref[0] = _dot(d2, ko_ref[...])


def kernel(x_, edges, W0, b0, W_enc, b_enc, K1, K2, Kd1, Kd2, Kout):
    xt = jnp.transpose(x_, (0, 2, 1, 3))
    b0r = b0.reshape(1, D)
    k1t = jnp.transpose(K1, (2, 1, 0))
    k2t = jnp.transpose(K2, (2, 1, 0))
    kd1t = jnp.transpose(Kd1, (2, 1, 0))
    kd2t = jnp.transpose(Kd2, (2, 1, 0))
    kot = Kout[:, :, 0].T

    whole = lambda shape: pl.BlockSpec(shape, lambda b: (0,) * len(shape))
    out = pl.pallas_call(
        _body,
        grid=(BATCH,),
        in_specs=[
            pl.BlockSpec((1, N, S, 2), lambda b: (b, 0, 0, 0)),
            whole((2, E)),
            whole((2, D)),
            whole((1, D)),
            whole((4, D, D)),
            whole((4, D)),
            whole((3, CIN, 256)),
            whole((3, 256, 256)),
            whole((3, 512, 256)),
            whole((3, 256 + CIN, 256)),
            whole((256, NCLS)),
        ],
        out_specs=pl.BlockSpec((1, S, NCLS), lambda b: (b, 0, 0)),
        out_shape=jax.ShapeDtypeStruct((BATCH, S, NCLS), F32),
        compiler_params=pltpu.CompilerParams(
            vmem_limit_bytes=100 * 1024 * 1024,
        ),
    )(xt, edges, W0, b0r, W_enc, b_enc, k1t, k2t, kd1t, kd2t, kot)
    return jnp.transpose(out, (0, 2, 1))


# speedup vs baseline: 3.2252x; 1.0072x over previous
import jax
import jax.numpy as jnp
from jax.experimental import pallas as pl
from jax.experimental.pallas import tpu as pltpu

D = 128
N = 32
BATCH = 4
S = 256
E = 256
NCLS = 10
CIN = N * D
F32 = jnp.float32
BF16 = jnp.bfloat16


def _relu(x):
    return jnp.maximum(x, 0.0)


def _dot(a, b):
    return jnp.dot(a, b, preferred_element_type=F32)


def _shift_down(p):
    return jnp.concatenate([jnp.zeros_like(p[:1]), p[:-1]], axis=0)


def _shift_up(p):
    return jnp.concatenate([p[1:], jnp.zeros_like(p[:1])], axis=0)


def _conv_s1(x, w0, w1, w2):
    return _shift_down(_dot(x, w0)) + _dot(x, w1) + _shift_up(_dot(x, w2))


def _conv_s2(x, w01, w2):
    L, C = x.shape
    v = x.reshape(L // 2, 2 * C)
    p01 = _dot(v, w01)
    p2 = _dot(v[:, :C], w2)
    return p01 + _shift_up(p2)


def _up2(x):
    L, C = x.shape
    return jnp.broadcast_to(x[:, None, :], (L, 2, C)).reshape(2 * L, C)


def _body(xt_ref, edg_ref, w0_ref, b0_ref, we_ref, be_ref,
          k1_ref, k1b_ref, k2_ref, kd1_ref, kd2_ref, ko_ref, out_ref):
    src = edg_ref[0:1, :]
    dst = edg_ref[1:2, :]
    ni = jax.lax.broadcasted_iota(jnp.int32, (N, E), 0)
    dst_oh = (ni == dst).astype(F32)
    src_oh = (ni == src).astype(F32)
    A = jax.lax.dot_general(dst_oh, src_oh, (((1,), (1,)), ((), ())),
                            preferred_element_type=F32)
    r = jax.lax.broadcasted_iota(jnp.int32, (N, N), 0)
    c = jax.lax.broadcasted_iota(jnp.int32, (N, N), 1)
    M = (A + (r == c).astype(F32)).astype(BF16)

    def _wmul(h3, w):
        return jax.lax.dot_general(h3, w, (((2,), (0,)), ((), ())),
                                   preferred_element_type=F32)

    def _mmul(m, z3):
        return jax.lax.dot_general(m, z3, (((1,), (0,)), ((), ())),
                                   preferred_element_type=F32)

    x0 = xt_ref[0]
    z = jax.lax.dot_general(x0, w0_ref[...], (((1,), (0,)), ((), ())),
                            preferred_element_type=F32)
    h = _relu(_mmul(M, z.astype(BF16)) + b0_ref[...].reshape(1, 1, D))
    for i in range(4):
        z = _wmul(h.astype(BF16), we_ref[i])
        h = _relu(_mmul(M, z.astype(BF16)) + be_ref[i:i + 1, :].reshape(1, 1, D))

    hc = jnp.swapaxes(h, 0, 1).reshape(S, CIN)

    e1 = _relu(_conv_s2(hc, k1_ref[...], k1b_ref[...]))
    e2 = _relu(_conv_s2(e1, k2_ref[0], k2_ref[1, :256]))
    u1 = _up2(e2)
    d1 = _relu(_conv_s1(u1, kd1_ref[0, :256], kd1_ref[1, :256], kd1_ref[2, :256])
               + _conv_s1(e1, kd1_ref[0, 256:], kd1_ref[1, 256:], kd1_ref[2, 256:]))
    u2 = _up2(d1)
    d2 = _relu(_conv_s1(u2, kd2_ref[0, :256], kd2_ref[1, :256], kd2_ref[2, :256])
               + _conv_s1(hc, kd2_ref[0, 256:], kd2_ref[1, 256:], kd2_ref[2, 256:]))
    out_ref[0] = _dot(d2, ko_ref[...])


def kernel(x_, edges, W0, b0, W_enc, b_enc, K1, K2, Kd1, Kd2, Kout):
    xt = jnp.transpose(x_, (0, 2, 3, 1))
    b0r = b0.reshape(1, D)
    web = W_enc.astype(BF16)
    k1t = jnp.transpose(K1, (2, 1, 0))
    k1m = jnp.concatenate([k1t[0], k1t[1]], axis=0)
    k2t = jnp.transpose(K2, (2, 1, 0))
    k2r = jnp.stack([jnp.concatenate([k2t[0], k2t[1]], axis=0),
                     jnp.pad(k2t[2], ((0, 256), (0, 0)))])
    kd1t = jnp.transpose(Kd1, (2, 1, 0))
    kd2t = jnp.transpose(Kd2, (2, 1, 0))
    kot = Kout[:, :, 0].T

    whole = lambda shape: pl.BlockSpec(shape, lambda b: (0,) * len(shape))
    out = pl.pallas_call(
        _body,
        grid=(BATCH,),
        in_specs=[
            pl.BlockSpec((1, N, 2, S), lambda b: (b, 0, 0, 0)),
            whole((2, E)),
            whole((2, D)),
            whole((1, D)),
            whole((4, D, D)),
            whole((4, D)),
            whole((CIN * 2, 256)),
            whole((CIN, 256)),
            whole((2, 512, 256)),
            whole((3, 512, 256)),
            whole((3, 256 + CIN, 256)),
            whole((256, NCLS)),
        ],
        out_specs=pl.BlockSpec((1, S, NCLS), lambda b: (b, 0, 0)),
        out_shape=jax.ShapeDtypeStruct((BATCH, S, NCLS), F32),
        compiler_params=pltpu.CompilerParams(
            vmem_limit_bytes=100 * 1024 * 1024,
        ),
    )(xt, edges, W0, b0r, web, b_enc, k1m, k1t[2], k2r, kd1t, kd2t, kot)
    return jnp.transpose(out, (0, 2, 1))
